# baseline (speedup 1.0000x reference)
import functools

import jax
import jax.numpy as jnp
from jax import lax
from jax.experimental import pallas as pl
from jax.experimental.pallas import tpu as pltpu
from jax.experimental.pallas import tpu_sc as plsc

B = 16384
D = 128
NC = 2
NS = 16
NW = NC * NS
BPW = B // NW
CH = 128
NCH = BPW // CH
GRP = 1
CPR = NCH // GRP
NSLOT = 3


def _sc_gather_body(ent_hbm, rel_hbm, src_hbm, out_hbm, src_v, rows_v,
                    sem0, sem1, sem2):
    sems = (sem0, sem1, sem2)
    wid = lax.axis_index("s") * NC + lax.axis_index("c")
    base = wid * BPW
    pltpu.sync_copy(src_hbm.at[:, pl.ds(wid * NCH, NCH)], src_v)

    tables = (ent_hbm, rel_hbm, ent_hbm)

    def gather(role, chunk):
        return pltpu.async_copy(
            tables[role].at[src_v.at[role, chunk]],
            rows_v.at[role], sems[role])

    def scatter(role, chunk):
        return pltpu.async_copy(
            rows_v.at[role],
            out_hbm.at[role, pl.ds(base + chunk * GRP * CH, GRP * CH)],
            sems[role])

    g = {(r, 0): gather(r, 0) for r in range(3)}
    s = {}
    for c in range(3 * CPR):
        role, chunk = c % 3, c // 3
        g[(role, chunk)].wait()
        s[(role, chunk)] = scatter(role, chunk)
        if chunk + 1 < CPR:
            s[(role, chunk)].wait()
            g[(role, chunk + 1)] = gather(role, chunk + 1)
    for r in range(3):
        s[(r, CPR - 1)].wait()


@jax.jit
def _run(entity_embedding, relation_embedding, src_idx):
    mesh = plsc.VectorSubcoreMesh(core_axis_name="c", subcore_axis_name="s")
    k = functools.partial(
        pl.kernel,
        out_type=jax.ShapeDtypeStruct((3, B, D), jnp.float32),
        mesh=mesh,
        scratch_types=[
            pltpu.VMEM((3, NCH, CH), jnp.int32),
            pltpu.VMEM((NSLOT, GRP * CH, D), jnp.float32),
            pltpu.SemaphoreType.DMA,
            pltpu.SemaphoreType.DMA,
            pltpu.SemaphoreType.DMA,
        ],
    )(_sc_gather_body)
    out3 = k(entity_embedding, relation_embedding, src_idx)
    return out3.transpose(1, 0, 2)


def kernel(sample, entity_embedding, relation_embedding):
    src_idx = sample.astype(jnp.int32).T.reshape(3, B // CH, CH)
    return _run(entity_embedding, relation_embedding, src_idx)


# speedup vs baseline: 1.0323x; 1.0323x over previous
import functools

import jax
import jax.numpy as jnp
from jax import lax
from jax.experimental import pallas as pl
from jax.experimental.pallas import tpu as pltpu
from jax.experimental.pallas import tpu_sc as plsc

B = 16384
D = 128
NC = 2
NS = 16
NW = NC * NS
BPW = B // NW
CH = 128
NCH = BPW // CH
NSLOT = 6


def _sc_gather_body(ent_hbm, rel_hbm, src_hbm, out_hbm, src_v, rows_v,
                    sem0, sem1, sem2, sem3, sem4, sem5):
    sems = (sem0, sem1, sem2, sem3, sem4, sem5)
    wid = lax.axis_index("s") * NC + lax.axis_index("c")
    base = wid * BPW
    pltpu.sync_copy(src_hbm.at[:, pl.ds(wid * NCH, NCH)], src_v)

    tables = (ent_hbm, rel_hbm, ent_hbm)

    def slot(role, chunk):
        return 2 * role + (chunk & 1)

    def gather(role, chunk):
        sl = slot(role, chunk)
        return pltpu.async_copy(
            tables[role].at[src_v.at[role, chunk]],
            rows_v.at[sl], sems[sl])

    def scatter(role, chunk):
        sl = slot(role, chunk)
        return pltpu.async_copy(
            rows_v.at[sl],
            out_hbm.at[role, pl.ds(base + chunk * CH, CH)],
            sems[sl])

    g = {(r, ch): gather(r, ch) for r in range(3) for ch in range(2)}
    s = {}
    for c in range(3 * NCH):
        role, chunk = c % 3, c // 3
        g[(role, chunk)].wait()
        s[(role, chunk)] = scatter(role, chunk)
        if 1 <= chunk < NCH - 1:
            s[(role, chunk - 1)].wait()
            g[(role, chunk + 1)] = gather(role, chunk + 1)
    for r in range(3):
        s[(r, NCH - 1)].wait()
        s[(r, NCH - 2)].wait()


@jax.jit
def _run(entity_embedding, relation_embedding, src_idx):
    mesh = plsc.VectorSubcoreMesh(core_axis_name="c", subcore_axis_name="s")
    k = functools.partial(
        pl.kernel,
        out_type=jax.ShapeDtypeStruct((3, B, D), jnp.float32),
        mesh=mesh,
        scratch_types=[
            pltpu.VMEM((3, NCH, CH), jnp.int32),
            pltpu.VMEM((NSLOT, CH, D), jnp.float32),
        ] + [pltpu.SemaphoreType.DMA] * NSLOT,
    )(_sc_gather_body)
    out3 = k(entity_embedding, relation_embedding, src_idx)
    return out3.transpose(1, 0, 2)


def kernel(sample, entity_embedding, relation_embedding):
    src_idx = sample.astype(jnp.int32).T.reshape(3, B // CH, CH)
    return _run(entity_embedding, relation_embedding, src_idx)
